# baseline (speedup 1.0000x reference)
import functools

import numpy as np

import jax
import jax.numpy as jnp
from jax import lax
from jax.experimental import pallas as pl
from jax.experimental.pallas import tpu as pltpu
from jax.experimental.pallas import tpu_sc as plsc

VOCAB = 100000
D = 768
B = 4
S = 2048
N = B * S
NC = 2
NS = 16
NW = NC * NS
TPW = N // NW
CH = 32
NCH = TPW // CH
LANES = 16
VEC = D // LANES
STRIPE = S * D // NS


@functools.lru_cache(maxsize=1)
def _positional_encoding(max_seq, d_model):
    pos = np.arange(max_seq, dtype=np.float32)[:, None]
    i = np.arange(0, d_model, 2, dtype=np.float32)[None, :]
    denom = np.power(np.float32(10000.0), i / np.float32(d_model))
    arg = (pos / denom).astype(np.float32)
    pe = np.stack([np.sin(arg), np.cos(arg)], axis=2).astype(np.float32)
    return jnp.asarray(pe.reshape(max_seq * d_model))


def _body(table, tokens, pe, out, idx_v,
          rows0, rows1, rows2, pe0, pe1,
          sg0, sg1, sg2, sp0, sp1, so0, so1, so2):
    rows = (rows0, rows1, rows2)
    pes = (pe0, pe1)
    sgs = (sg0, sg1, sg2)
    sps = (sp0, sp1)
    sos = (so0, so1, so2)
    sid = lax.axis_index("s")
    wid = sid * NC + lax.axis_index("c")
    base = wid * TPW
    s0 = lax.rem(base, S)

    pltpu.sync_copy(tokens.at[pl.ds(base, TPW)], idx_v)

    def start_gather(c):
        ir = c % 3
        pltpu.async_copy(table.at[idx_v.at[pl.ds(c * CH, CH)]],
                         rows[ir], sgs[ir])

    def start_pe(c):
        ip = c % 2
        pltpu.async_copy(pe.at[pl.ds((s0 + c * CH) * D, CH * D)],
                         pes[ip], sps[ip])

    start_gather(0)
    start_gather(1)
    start_pe(0)
    start_pe(1)
    for c in range(NCH):
        ir, ip = c % 3, c % 2
        cb = c * CH
        pltpu.make_async_copy(table.at[idx_v.at[pl.ds(cb, CH)]],
                              rows[ir], sgs[ir]).wait()
        pltpu.make_async_copy(pe.at[pl.ds((s0 + cb) * D, CH * D)],
                              pes[ip], sps[ip]).wait()

        rv, pv = rows[ir], pes[ip]

        @plsc.parallel_loop(0, CH, step=1, unroll=2)
        def _add(t):
            tD = t * D
            for j in range(VEC):
                rv[t, pl.ds(j * LANES, LANES)] = (
                    rv[t, pl.ds(j * LANES, LANES)]
                    + pv[pl.ds(tD + j * LANES, LANES)])

        pltpu.async_copy(rows[ir], out.at[pl.ds(base + cb, CH)], sos[ir])
        n = c + 2
        if n < NCH:
            jr = n % 3
            if c >= 1:
                pltpu.make_async_copy(
                    rows[jr], out.at[pl.ds(base + (c - 1) * CH, CH)],
                    sos[jr]).wait()
            start_gather(n)
            start_pe(n)
    for k in range(3):
        c = NCH - 3 + k
        pltpu.make_async_copy(rows[c % 3], out.at[pl.ds(base + c * CH, CH)],
                              sos[c % 3]).wait()


@jax.jit
def kernel(tokens, table):
    pe = _positional_encoding(S, D)
    tok = tokens.reshape(N).astype(jnp.int32)
    mesh = plsc.VectorSubcoreMesh(core_axis_name="c", subcore_axis_name="s")
    f = pl.kernel(
        _body,
        out_type=jax.ShapeDtypeStruct((N, D), jnp.float32),
        mesh=mesh,
        scratch_types=[
            pltpu.VMEM((TPW,), jnp.int32),
            pltpu.VMEM((CH, D), jnp.float32),
            pltpu.VMEM((CH, D), jnp.float32),
            pltpu.VMEM((CH, D), jnp.float32),
            pltpu.VMEM((CH * D,), jnp.float32),
            pltpu.VMEM((CH * D,), jnp.float32),
            pltpu.SemaphoreType.DMA,
            pltpu.SemaphoreType.DMA,
            pltpu.SemaphoreType.DMA,
            pltpu.SemaphoreType.DMA,
            pltpu.SemaphoreType.DMA,
            pltpu.SemaphoreType.DMA,
            pltpu.SemaphoreType.DMA,
            pltpu.SemaphoreType.DMA,
        ],
    )
    out = f(table, tok, pe)
    return out.reshape(B, S, D)


# speedup vs baseline: 1.1118x; 1.1118x over previous
import functools

import ml_dtypes
import numpy as np

import jax
import jax.numpy as jnp
from jax import lax
from jax.experimental import pallas as pl
from jax.experimental.pallas import tpu as pltpu
from jax.experimental.pallas import tpu_sc as plsc

VOCAB = 100000
D = 768
B = 4
S = 2048
N = B * S
NC = 2
NS = 16
NW = NC * NS
TPW = N // NW
CH = 8
NCH = TPW // CH
NB = 4
LANES = 16
VEC = D // LANES


@functools.lru_cache(maxsize=1)
def _positional_encoding(max_seq, d_model):
    pos = np.arange(max_seq, dtype=np.float32)[:, None]
    i = np.arange(0, d_model, 2, dtype=np.float32)[None, :]
    denom = np.power(np.float32(10000.0), i / np.float32(d_model))
    arg = (pos / denom).astype(np.float32)
    pe = np.stack([np.sin(arg), np.cos(arg)], axis=2).astype(np.float32)
    x = pe.reshape(-1, 2, LANES)
    shuf = np.stack([x[:, 0, :], x[:, 1, :]], axis=-1).reshape(-1)
    bf = shuf.astype(ml_dtypes.bfloat16)
    return jnp.asarray(bf.view(np.int32))


def _body(table, tokens, pe, out, idx_v,
          rows0, rows1, rows2, rows3, pe_t,
          sg0, sg1, sg2, sg3, so0, so1, so2, so3, spe):
    rows = (rows0, rows1, rows2, rows3)
    sgs = (sg0, sg1, sg2, sg3)
    sos = (so0, so1, so2, so3)
    sid = lax.axis_index("s")
    wid = sid * NC + lax.axis_index("c")
    base = wid * TPW
    s0 = lax.rem(base, S)

    pe_load = pltpu.async_copy(pe.at[pl.ds(s0 * (D // 2), TPW * (D // 2))],
                               pe_t, spe)
    pltpu.sync_copy(tokens.at[pl.ds(base, TPW)], idx_v)

    def start_gather(c):
        pltpu.async_copy(table.at[idx_v.at[pl.ds(c * CH, CH)]],
                         rows[c % NB], sgs[c % NB])

    start_gather(0)
    start_gather(1)
    pe_load.wait()

    def chunk_step(c, b):
        cb = c * CH
        pltpu.make_async_copy(table.at[idx_v.at[pl.ds(cb, CH)]],
                              rows[b], sgs[b]).wait()
        rv = rows[b]

        @plsc.parallel_loop(0, CH, step=1, unroll=2)
        def _add(t):
            rbase = (cb + t) * (D // 2)
            for j in range(VEC // 2):
                pv32 = pe_t[pl.ds(rbase + j * LANES, LANES)]
                pv = plsc.bitcast(pv32, jnp.bfloat16)
                a0, a1 = plsc.unpack(pv, format=plsc.PackFormat.INTERLEAVED)
                sl0 = (t, pl.ds((2 * j) * LANES, LANES))
                sl1 = (t, pl.ds((2 * j + 1) * LANES, LANES))
                rv[sl0] = rv[sl0] + a0
                rv[sl1] = rv[sl1] + a1

        pltpu.async_copy(rv, out.at[pl.ds(base + cb, CH)], sos[b])
        n = c + 2
        bn = (b + 2) % NB

        @pl.when(n < NCH)
        def _prefetch():
            @pl.when(c >= 2)
            def _drain():
                pltpu.make_async_copy(
                    rows[bn], out.at[pl.ds(base + (c - 2) * CH, CH)],
                    sos[bn]).wait()
            pltpu.async_copy(table.at[idx_v.at[pl.ds(n * CH, CH)]],
                             rows[bn], sgs[bn])

    def group(g, carry):
        for b in range(NB):
            chunk_step(g * NB + b, b)
        return carry

    lax.fori_loop(0, NCH // NB, group, 0)
    for k in range(NB):
        c = NCH - NB + k
        pltpu.make_async_copy(rows[c % NB], out.at[pl.ds(base + c * CH, CH)],
                              sos[c % NB]).wait()


@jax.jit
def kernel(tokens, table):
    pe = _positional_encoding(S, D)
    tok = tokens.reshape(N).astype(jnp.int32)
    mesh = plsc.VectorSubcoreMesh(core_axis_name="c", subcore_axis_name="s")
    f = pl.kernel(
        _body,
        out_type=jax.ShapeDtypeStruct((N, D), jnp.float32),
        mesh=mesh,
        compiler_params=pltpu.CompilerParams(needs_layout_passes=False),
        scratch_types=[
            pltpu.VMEM((TPW,), jnp.int32),
            pltpu.VMEM((CH, D), jnp.float32),
            pltpu.VMEM((CH, D), jnp.float32),
            pltpu.VMEM((CH, D), jnp.float32),
            pltpu.VMEM((CH, D), jnp.float32),
            pltpu.VMEM((TPW * D // 2,), jnp.int32),
            pltpu.SemaphoreType.DMA,
            pltpu.SemaphoreType.DMA,
            pltpu.SemaphoreType.DMA,
            pltpu.SemaphoreType.DMA,
            pltpu.SemaphoreType.DMA,
            pltpu.SemaphoreType.DMA,
            pltpu.SemaphoreType.DMA,
            pltpu.SemaphoreType.DMA,
            pltpu.SemaphoreType.DMA,
        ],
    )
    out = f(table, tok, pe)
    return out.reshape(B, S, D)


# speedup vs baseline: 1.1164x; 1.0041x over previous
import functools

import ml_dtypes
import numpy as np

import jax
import jax.numpy as jnp
from jax import lax
from jax.experimental import pallas as pl
from jax.experimental.pallas import tpu as pltpu
from jax.experimental.pallas import tpu_sc as plsc

VOCAB = 100000
D = 768
B = 4
S = 2048
N = B * S
NC = 2
NS = 16
NW = NC * NS
TPW = N // NW
CH = 8
NCH = TPW // CH
NB = 4
LANES = 16
VEC = D // LANES


@functools.lru_cache(maxsize=1)
def _positional_encoding(max_seq, d_model):
    pos = np.arange(max_seq, dtype=np.float32)[:, None]
    i = np.arange(0, d_model, 2, dtype=np.float32)[None, :]
    denom = np.power(np.float32(10000.0), i / np.float32(d_model))
    arg = (pos / denom).astype(np.float32)
    pe = np.stack([np.sin(arg), np.cos(arg)], axis=2).astype(np.float32)
    x = pe.reshape(-1, 2, LANES)
    shuf = np.stack([x[:, 0, :], x[:, 1, :]], axis=-1).reshape(-1)
    bf = shuf.astype(ml_dtypes.bfloat16)
    return jnp.asarray(bf.view(np.int32))


def _body(table, tokens, pe, out, idx_v,
          rows0, rows1, rows2, rows3, pe_t,
          sg0, sg1, sg2, sg3, so0, so1, so2, so3, spe):
    rows = (rows0, rows1, rows2, rows3)
    sgs = (sg0, sg1, sg2, sg3)
    sos = (so0, so1, so2, so3)
    sid = lax.axis_index("s")
    wid = sid * NC + lax.axis_index("c")
    base = wid * TPW
    s0 = lax.rem(base, S)

    pe_load = pltpu.async_copy(pe.at[pl.ds(s0 * (D // 2), TPW * (D // 2))],
                               pe_t, spe)
    pltpu.sync_copy(tokens.at[pl.ds(base, TPW)], idx_v)

    def start_gather(c):
        pltpu.async_copy(table.at[idx_v.at[pl.ds(c * CH, CH)]],
                         rows[c % NB], sgs[c % NB])

    start_gather(0)
    start_gather(1)
    pe_load.wait()

    def chunk_step(c, b):
        cb = c * CH
        pltpu.make_async_copy(table.at[idx_v.at[pl.ds(cb, CH)]],
                              rows[b], sgs[b]).wait()
        rv = rows[b]

        @plsc.parallel_loop(0, CH, step=1, unroll=2)
        def _add(t):
            rbase = (cb + t) * (D // 2)
            for j in range(VEC // 2):
                pv32 = pe_t[pl.ds(rbase + j * LANES, LANES)]
                a0 = plsc.bitcast(pv32 << 16, jnp.float32)
                a1 = plsc.bitcast(pv32 & jnp.int32(-65536), jnp.float32)
                sl0 = (t, pl.ds((2 * j) * LANES, LANES))
                sl1 = (t, pl.ds((2 * j + 1) * LANES, LANES))
                rv[sl0] = rv[sl0] + a0
                rv[sl1] = rv[sl1] + a1

        pltpu.async_copy(rv, out.at[pl.ds(base + cb, CH)], sos[b])
        n = c + 2
        bn = (b + 2) % NB

        @pl.when(n < NCH)
        def _prefetch():
            @pl.when(c >= 2)
            def _drain():
                pltpu.make_async_copy(
                    rows[bn], out.at[pl.ds(base + (c - 2) * CH, CH)],
                    sos[bn]).wait()
            pltpu.async_copy(table.at[idx_v.at[pl.ds(n * CH, CH)]],
                             rows[bn], sgs[bn])

    def group(g, carry):
        for b in range(NB):
            chunk_step(g * NB + b, b)
        return carry

    lax.fori_loop(0, NCH // NB, group, 0)
    for k in range(NB):
        c = NCH - NB + k
        pltpu.make_async_copy(rows[c % NB], out.at[pl.ds(base + c * CH, CH)],
                              sos[c % NB]).wait()


@jax.jit
def kernel(tokens, table):
    pe = _positional_encoding(S, D)
    tok = tokens.reshape(N).astype(jnp.int32)
    mesh = plsc.VectorSubcoreMesh(core_axis_name="c", subcore_axis_name="s")
    f = pl.kernel(
        _body,
        out_type=jax.ShapeDtypeStruct((N, D), jnp.float32),
        mesh=mesh,
        compiler_params=pltpu.CompilerParams(needs_layout_passes=False),
        scratch_types=[
            pltpu.VMEM((TPW,), jnp.int32),
            pltpu.VMEM((CH, D), jnp.float32),
            pltpu.VMEM((CH, D), jnp.float32),
            pltpu.VMEM((CH, D), jnp.float32),
            pltpu.VMEM((CH, D), jnp.float32),
            pltpu.VMEM((TPW * D // 2,), jnp.int32),
            pltpu.SemaphoreType.DMA,
            pltpu.SemaphoreType.DMA,
            pltpu.SemaphoreType.DMA,
            pltpu.SemaphoreType.DMA,
            pltpu.SemaphoreType.DMA,
            pltpu.SemaphoreType.DMA,
            pltpu.SemaphoreType.DMA,
            pltpu.SemaphoreType.DMA,
            pltpu.SemaphoreType.DMA,
        ],
    )
    out = f(table, tok, pe)
    return out.reshape(B, S, D)


# speedup vs baseline: 1.2606x; 1.1292x over previous
import functools

import ml_dtypes
import numpy as np

import jax
import jax.numpy as jnp
from jax import lax
from jax.experimental import pallas as pl
from jax.experimental.pallas import tpu as pltpu
from jax.experimental.pallas import tpu_sc as plsc

VOCAB = 100000
D = 768
B = 4
S = 2048
N = B * S
NC = 2
NS = 16
NW = NC * NS
TPW = N // NW
CH = 16
NCH = TPW // CH
NB = 2
LANES = 16
VEC = D // LANES


@functools.lru_cache(maxsize=1)
def _positional_encoding(max_seq, d_model):
    pos = np.arange(max_seq, dtype=np.float32)[:, None]
    i = np.arange(0, d_model, 2, dtype=np.float32)[None, :]
    denom = np.power(np.float32(10000.0), i / np.float32(d_model))
    arg = (pos / denom).astype(np.float32)
    pe = np.stack([np.sin(arg), np.cos(arg)], axis=2).astype(np.float32)
    x = pe.reshape(-1, 2, LANES)
    shuf = np.stack([x[:, 0, :], x[:, 1, :]], axis=-1).reshape(-1)
    bf = shuf.astype(ml_dtypes.bfloat16)
    return jnp.asarray(bf.view(np.int32))


def _body(table, tokens, pe, out, idx_v,
          rows0, rows1, pe_t,
          sg0, sg1, so0, so1, spe):
    rows = (rows0, rows1)
    sgs = (sg0, sg1)
    sos = (so0, so1)
    sid = lax.axis_index("s")
    wid = sid * NC + lax.axis_index("c")
    base = wid * TPW
    s0 = lax.rem(base, S)

    pe_load = pltpu.async_copy(pe.at[pl.ds(s0 * (D // 2), TPW * (D // 2))],
                               pe_t, spe)
    pltpu.sync_copy(tokens.at[pl.ds(base, TPW)], idx_v)

    def start_gather(c):
        pltpu.async_copy(table.at[idx_v.at[pl.ds(c * CH, CH)]],
                         rows[c % NB], sgs[c % NB])

    start_gather(0)
    start_gather(1)
    pe_load.wait()

    def chunk_step(c, b):
        cb = c * CH
        pltpu.make_async_copy(table.at[idx_v.at[pl.ds(cb, CH)]],
                              rows[b], sgs[b]).wait()
        rv = rows[b]

        @plsc.parallel_loop(0, CH, step=1, unroll=2)
        def _add(t):
            rbase = (cb + t) * (D // 2)
            for j in range(VEC // 2):
                pv32 = pe_t[pl.ds(rbase + j * LANES, LANES)]
                a0 = plsc.bitcast(pv32 << 16, jnp.float32)
                a1 = plsc.bitcast(pv32 & jnp.int32(-65536), jnp.float32)
                sl0 = (t, pl.ds((2 * j) * LANES, LANES))
                sl1 = (t, pl.ds((2 * j + 1) * LANES, LANES))
                rv[sl0] = rv[sl0] + a0
                rv[sl1] = rv[sl1] + a1

        pltpu.async_copy(rv, out.at[pl.ds(base + cb, CH)], sos[b])
        n = c + 2
        bn = (b + 2) % NB

        @pl.when(n < NCH)
        def _prefetch():
            @pl.when(c >= 2)
            def _drain():
                pltpu.make_async_copy(
                    rows[bn], out.at[pl.ds(base + (c - 2) * CH, CH)],
                    sos[bn]).wait()
            pltpu.async_copy(table.at[idx_v.at[pl.ds(n * CH, CH)]],
                             rows[bn], sgs[bn])

    def group(g, carry):
        for b in range(NB):
            chunk_step(g * NB + b, b)
        return carry

    lax.fori_loop(0, NCH // NB, group, 0)
    for k in range(NB):
        c = NCH - NB + k
        pltpu.make_async_copy(rows[c % NB], out.at[pl.ds(base + c * CH, CH)],
                              sos[c % NB]).wait()


@jax.jit
def kernel(tokens, table):
    pe = _positional_encoding(S, D)
    tok = tokens.reshape(N).astype(jnp.int32)
    mesh = plsc.VectorSubcoreMesh(core_axis_name="c", subcore_axis_name="s")
    f = pl.kernel(
        _body,
        out_type=jax.ShapeDtypeStruct((N, D), jnp.float32),
        mesh=mesh,
        compiler_params=pltpu.CompilerParams(needs_layout_passes=False),
        scratch_types=[
            pltpu.VMEM((TPW,), jnp.int32),
            pltpu.VMEM((CH, D), jnp.float32),
            pltpu.VMEM((CH, D), jnp.float32),
            pltpu.VMEM((TPW * D // 2,), jnp.int32),
            pltpu.SemaphoreType.DMA,
            pltpu.SemaphoreType.DMA,
            pltpu.SemaphoreType.DMA,
            pltpu.SemaphoreType.DMA,
            pltpu.SemaphoreType.DMA,
        ],
    )
    out = f(table, tok, pe)
    return out.reshape(B, S, D)


# speedup vs baseline: 1.2654x; 1.0038x over previous
import functools

import ml_dtypes
import numpy as np

import jax
import jax.numpy as jnp
from jax import lax
from jax.experimental import layout as jex_layout
from jax.experimental import pallas as pl
from jax.experimental.pallas import tpu as pltpu
from jax.experimental.pallas import tpu_sc as plsc

VOCAB = 100000
D = 768
B = 4
S = 2048
N = B * S
NC = 2
NS = 16
NW = NC * NS
TPW = N // NW
CH = 16
NCH = TPW // CH
NB = 2
LANES = 16
VEC = D // LANES


@functools.lru_cache(maxsize=1)
def _positional_encoding(max_seq, d_model):
    pos = np.arange(max_seq, dtype=np.float32)[:, None]
    i = np.arange(0, d_model, 2, dtype=np.float32)[None, :]
    denom = np.power(np.float32(10000.0), i / np.float32(d_model))
    arg = (pos / denom).astype(np.float32)
    pe = np.stack([np.sin(arg), np.cos(arg)], axis=2).astype(np.float32)
    x = pe.reshape(-1, 2, LANES)
    shuf = np.stack([x[:, 0, :], x[:, 1, :]], axis=-1).reshape(-1)
    bf = shuf.astype(ml_dtypes.bfloat16)
    return jnp.asarray(bf.view(np.int32))


def _body(table, tokens, pe, out, idx_v,
          rows0, rows1, pe_t,
          sg0, sg1, so0, so1, spe):
    rows = (rows0, rows1)
    sgs = (sg0, sg1)
    sos = (so0, so1)
    sid = lax.axis_index("s")
    wid = sid * NC + lax.axis_index("c")
    base = wid * TPW
    s0 = lax.rem(base, S)

    pe_load = pltpu.async_copy(pe.at[pl.ds(s0 * (D // 2), TPW * (D // 2))],
                               pe_t, spe)
    pltpu.sync_copy(tokens.at[pl.ds(base, TPW)], idx_v)

    def start_gather(c):
        pltpu.async_copy(table.at[idx_v.at[pl.ds(c * CH, CH)]],
                         rows[c % NB], sgs[c % NB])

    start_gather(0)
    start_gather(1)
    pe_load.wait()

    def chunk_step(c, b):
        cb = c * CH
        pltpu.make_async_copy(table.at[idx_v.at[pl.ds(cb, CH)]],
                              rows[b], sgs[b]).wait()
        rv = rows[b]

        @plsc.parallel_loop(0, CH, step=1, unroll=2)
        def _add(t):
            rbase = (cb + t) * (D // 2)
            for j in range(VEC // 2):
                pv32 = pe_t[pl.ds(rbase + j * LANES, LANES)]
                a0 = plsc.bitcast(pv32 << 16, jnp.float32)
                a1 = plsc.bitcast(pv32 & jnp.int32(-65536), jnp.float32)
                sl0 = (t, pl.ds((2 * j) * LANES, LANES))
                sl1 = (t, pl.ds((2 * j + 1) * LANES, LANES))
                rv[sl0] = rv[sl0] + a0
                rv[sl1] = rv[sl1] + a1

        pltpu.async_copy(rv, out.at[pl.ds(base + cb, CH)], sos[b])
        n = c + 2
        bn = (b + 2) % NB

        @pl.when(n < NCH)
        def _prefetch():
            @pl.when(c >= 2)
            def _drain():
                pltpu.make_async_copy(
                    rows[bn], out.at[pl.ds(base + (c - 2) * CH, CH)],
                    sos[bn]).wait()
            pltpu.async_copy(table.at[idx_v.at[pl.ds(n * CH, CH)]],
                             rows[bn], sgs[bn])

    def group(g, carry):
        for b in range(NB):
            chunk_step(g * NB + b, b)
        return carry

    lax.fori_loop(0, NCH // NB, group, 0)
    for k in range(NB):
        c = NCH - NB + k
        pltpu.make_async_copy(rows[c % NB], out.at[pl.ds(base + c * CH, CH)],
                              sos[c % NB]).wait()


@functools.lru_cache(maxsize=1)
def _jitted_kernel():
    fmt = jex_layout.Format(
        jex_layout.Layout(major_to_minor=(0, 1, 2), tiling=()),
        jax.sharding.SingleDeviceSharding(jax.devices()[0]))
    return jax.jit(_kernel_impl, out_shardings=fmt)


def kernel(tokens, table):
    return _jitted_kernel()(tokens, table)


def _kernel_impl(tokens, table):
    pe = _positional_encoding(S, D)
    tok = tokens.reshape(N).astype(jnp.int32)
    mesh = plsc.VectorSubcoreMesh(core_axis_name="c", subcore_axis_name="s")
    f = pl.kernel(
        _body,
        out_type=jax.ShapeDtypeStruct((N, D), jnp.float32),
        mesh=mesh,
        compiler_params=pltpu.CompilerParams(needs_layout_passes=False),
        scratch_types=[
            pltpu.VMEM((TPW,), jnp.int32),
            pltpu.VMEM((CH, D), jnp.float32),
            pltpu.VMEM((CH, D), jnp.float32),
            pltpu.VMEM((TPW * D // 2,), jnp.int32),
            pltpu.SemaphoreType.DMA,
            pltpu.SemaphoreType.DMA,
            pltpu.SemaphoreType.DMA,
            pltpu.SemaphoreType.DMA,
            pltpu.SemaphoreType.DMA,
        ],
    )
    out = f(table, tok, pe)
    return out.reshape(B, S, D)
